# initial kernel scaffold (unmeasured)
import jax
import jax.numpy as jnp
from jax import lax
from jax.experimental import pallas as pl
from jax.experimental.pallas import tpu as pltpu

N_DEV = 4
SQ = 256
SKV_SHARD = 4096
HQ = 8
DH = 128
DM = HQ * DH
SCALE = 0.08838834764831843
NEG = -1e9


def kernel(x, Wq, K_ext, V_ext, Wo):
    def body(x_ref, wq_ref, k_ref, v_ref, wo_ref, out_ref,
             comm_o, comm_s, ctx_ref,
             send_o, recv_o, send_s, recv_s):
        my_pos = lax.axis_index("i")
        left = lax.rem(my_pos + N_DEV - 1, N_DEV)
        right = lax.rem(my_pos + 1, N_DEV)

        barrier_sem = pltpu.get_barrier_semaphore()
        for nbr in (left, right):
            pl.semaphore_signal(
                barrier_sem, inc=1,
                device_id=(nbr,), device_id_type=pl.DeviceIdType.MESH,
            )
        pl.semaphore_wait(barrier_sem, 2)

        q = jnp.dot(x_ref[0], wq_ref[:, :],
                    preferred_element_type=jnp.float32)

        q_idx = lax.broadcasted_iota(jnp.int32, (SQ, SKV_SHARD), 0)
        k_idx = lax.broadcasted_iota(jnp.int32, (SQ, SKV_SHARD), 1)
        mask = ((q_idx // 64) % 4) == ((k_idx // 64) % 4)

        for h in range(HQ):
            qh = q[:, h * DH:(h + 1) * DH]
            kh = k_ref[0, :, h, :]
            vh = v_ref[0, :, h, :]
            s = lax.dot_general(
                qh, kh, (((1,), (1,)), ((), ())),
                preferred_element_type=jnp.float32) * SCALE
            s = jnp.where(mask, s, NEG)
            m = jnp.max(s, axis=1, keepdims=True)
            w = jnp.exp(s - m)
            l = jnp.sum(w, axis=1, keepdims=True)
            o = jnp.dot(w, vh, preferred_element_type=jnp.float32)
            comm_o[0, :, h * DH:(h + 1) * DH] = o
            comm_s[0, :, h:h + 1] = m
            comm_s[0, :, HQ + h:HQ + h + 1] = l

        for hop in range(N_DEV - 1):
            rdma_o = pltpu.make_async_remote_copy(
                src_ref=comm_o.at[hop], dst_ref=comm_o.at[hop + 1],
                send_sem=send_o.at[hop], recv_sem=recv_o.at[hop],
                device_id=(right,), device_id_type=pl.DeviceIdType.MESH,
            )
            rdma_s = pltpu.make_async_remote_copy(
                src_ref=comm_s.at[hop], dst_ref=comm_s.at[hop + 1],
                send_sem=send_s.at[hop], recv_sem=recv_s.at[hop],
                device_id=(right,), device_id_type=pl.DeviceIdType.MESH,
            )
            rdma_o.start()
            rdma_s.start()
            rdma_o.wait()
            rdma_s.wait()

        m_all = [comm_s[slot, :, 0:HQ] for slot in range(N_DEV)]
        l_all = [comm_s[slot, :, HQ:2 * HQ] for slot in range(N_DEV)]
        m_g = jnp.maximum(jnp.maximum(m_all[0], m_all[1]),
                          jnp.maximum(m_all[2], m_all[3]))
        scales = [jnp.exp(m_all[s] - m_g) for s in range(N_DEV)]
        l_g = sum(l_all[s] * scales[s] for s in range(N_DEV))

        for h in range(HQ):
            acc = sum(
                comm_o[s, :, h * DH:(h + 1) * DH] * scales[s][:, h:h + 1]
                for s in range(N_DEV)
            )
            ctx_ref[:, h * DH:(h + 1) * DH] = acc / l_g[:, h:h + 1]

        out_ref[0] = jnp.dot(ctx_ref[:, :], wo_ref[:, :],
                             preferred_element_type=jnp.float32)

    return pl.pallas_call(
        body,
        out_shape=jax.ShapeDtypeStruct((1, SQ, DM), jnp.float32),
        in_specs=[pl.BlockSpec(memory_space=pltpu.VMEM)] * 5,
        out_specs=pl.BlockSpec(memory_space=pltpu.VMEM),
        scratch_shapes=[
            pltpu.VMEM((N_DEV, SQ, DM), jnp.float32),
            pltpu.VMEM((N_DEV, SQ, 2 * HQ), jnp.float32),
            pltpu.VMEM((SQ, DM), jnp.float32),
            pltpu.SemaphoreType.DMA((N_DEV - 1,)),
            pltpu.SemaphoreType.DMA((N_DEV - 1,)),
            pltpu.SemaphoreType.DMA((N_DEV - 1,)),
            pltpu.SemaphoreType.DMA((N_DEV - 1,)),
        ],
        compiler_params=pltpu.CompilerParams(collective_id=0),
    )(x, Wq, K_ext, V_ext, Wo)


# baseline (device time: 99145 ns/iter reference)
import jax
import jax.numpy as jnp
from jax import lax
from jax.experimental import pallas as pl
from jax.experimental.pallas import tpu as pltpu

N_DEV = 4
SQ = 256
SKV_SHARD = 4096
HQ = 8
DH = 128
DM = HQ * DH
SCALE = 0.08838834764831843
NEG = -1e9


def kernel(x, Wq, K_ext, V_ext, Wo):
    def body(x_ref, wq_ref, k_ref, v_ref, wo_ref, out_ref,
             comm_o, comm_s, ctx_ref,
             send_o, recv_o, send_s, recv_s):
        my_pos = lax.axis_index("i")
        left = lax.rem(my_pos + N_DEV - 1, N_DEV)
        right = lax.rem(my_pos + 1, N_DEV)

        barrier_sem = pltpu.get_barrier_semaphore()
        for nbr in (left, right):
            pl.semaphore_signal(
                barrier_sem, inc=1,
                device_id=(nbr,), device_id_type=pl.DeviceIdType.MESH,
            )
        pl.semaphore_wait(barrier_sem, 2)

        q = jnp.dot(x_ref[0], wq_ref[:, :],
                    preferred_element_type=jnp.float32)

        q_idx = lax.broadcasted_iota(jnp.int32, (SQ, SKV_SHARD), 0)
        k_idx = lax.broadcasted_iota(jnp.int32, (SQ, SKV_SHARD), 1)
        mask = ((q_idx // 64) % 4) == ((k_idx // 64) % 4)

        for h in range(HQ):
            qh = q[:, h * DH:(h + 1) * DH]
            kh = k_ref[0, :, h, :]
            vh = v_ref[0, :, h, :]
            s = lax.dot_general(
                qh, kh, (((1,), (1,)), ((), ())),
                preferred_element_type=jnp.float32) * SCALE
            s = jnp.where(mask, s, NEG)
            m = jnp.max(s, axis=1, keepdims=True)
            w = jnp.exp(s - m)
            l = jnp.sum(w, axis=1, keepdims=True)
            o = jnp.dot(w, vh, preferred_element_type=jnp.float32)
            comm_o[0, :, h * DH:(h + 1) * DH] = o
            comm_s[0, :, h:h + 1] = m
            comm_s[0, :, HQ + h:HQ + h + 1] = l

        for hop in range(N_DEV - 1):
            rdma_o = pltpu.make_async_remote_copy(
                src_ref=comm_o.at[hop], dst_ref=comm_o.at[hop + 1],
                send_sem=send_o.at[hop], recv_sem=recv_o.at[hop],
                device_id=(right,), device_id_type=pl.DeviceIdType.MESH,
            )
            rdma_s = pltpu.make_async_remote_copy(
                src_ref=comm_s.at[hop], dst_ref=comm_s.at[hop + 1],
                send_sem=send_s.at[hop], recv_sem=recv_s.at[hop],
                device_id=(right,), device_id_type=pl.DeviceIdType.MESH,
            )
            rdma_o.start()
            rdma_s.start()
            rdma_o.wait()
            rdma_s.wait()

        m_all = [comm_s[slot, :, 0:HQ] for slot in range(N_DEV)]
        l_all = [comm_s[slot, :, HQ:2 * HQ] for slot in range(N_DEV)]
        m_g = jnp.maximum(jnp.maximum(m_all[0], m_all[1]),
                          jnp.maximum(m_all[2], m_all[3]))
        scales = [jnp.exp(m_all[s] - m_g) for s in range(N_DEV)]
        l_g = sum(l_all[s] * scales[s] for s in range(N_DEV))

        for h in range(HQ):
            acc = sum(
                comm_o[s, :, h * DH:(h + 1) * DH] * scales[s][:, h:h + 1]
                for s in range(N_DEV)
            )
            ctx_ref[:, h * DH:(h + 1) * DH] = acc / l_g[:, h:h + 1]

        out_ref[0] = jnp.dot(ctx_ref[:, :], wo_ref[:, :],
                             preferred_element_type=jnp.float32)

    return pl.pallas_call(
        body,
        out_shape=jax.ShapeDtypeStruct((1, SQ, DM), jnp.float32),
        in_specs=[pl.BlockSpec(memory_space=pltpu.VMEM)] * 5,
        out_specs=pl.BlockSpec(memory_space=pltpu.VMEM),
        scratch_shapes=[
            pltpu.VMEM((N_DEV, SQ, DM), jnp.float32),
            pltpu.VMEM((N_DEV, SQ, 2 * HQ), jnp.float32),
            pltpu.VMEM((SQ, DM), jnp.float32),
            pltpu.SemaphoreType.DMA((N_DEV - 1,)),
            pltpu.SemaphoreType.DMA((N_DEV - 1,)),
            pltpu.SemaphoreType.DMA((N_DEV - 1,)),
            pltpu.SemaphoreType.DMA((N_DEV - 1,)),
        ],
        compiler_params=pltpu.CompilerParams(
            collective_id=0,
            vmem_limit_bytes=100 * 1024 * 1024,
        ),
    )(x, Wq, K_ext, V_ext, Wo)


# device time: 63629 ns/iter; 1.5582x vs baseline; 1.5582x over previous
import jax
import jax.numpy as jnp
from jax import lax
from jax.experimental import pallas as pl
from jax.experimental.pallas import tpu as pltpu

N_DEV = 4
SQ = 256
SKV_SHARD = 4096
HQ = 8
DH = 128
DM = HQ * DH
CW = DM + DH
SCALE = 0.08838834764831843
NEG = -1e9
HALF = SQ // 2


def kernel(x, Wq, K_ext, V_ext, Wo):
    def body(x_ref, wq_ref, k_ref, v_ref, wo_ref, out_ref,
             comm, ctx_ref, ss, rs):
        my_pos = lax.axis_index("i")
        left = lax.rem(my_pos + N_DEV - 1, N_DEV)
        right = lax.rem(my_pos + 1, N_DEV)

        barrier_sem = pltpu.get_barrier_semaphore()
        for nbr in (left, right):
            pl.semaphore_signal(
                barrier_sem, inc=1,
                device_id=(nbr,), device_id_type=pl.DeviceIdType.MESH,
            )
        pl.semaphore_wait(barrier_sem, 2)

        q = jnp.dot(x_ref[0], wq_ref[:, :],
                    preferred_element_type=jnp.float32) * SCALE

        q_idx = lax.broadcasted_iota(jnp.int32, (SQ, SKV_SHARD), 0)
        k_idx = lax.broadcasted_iota(jnp.int32, (SQ, SKV_SHARD), 1)
        bias = jnp.where(((q_idx // 64) % 4) == ((k_idx // 64) % 4),
                         0.0, NEG).astype(jnp.float32)

        for h in range(HQ):
            qh = q[:, h * DH:(h + 1) * DH]
            kh = k_ref[0, :, h, :]
            vh = v_ref[0, :, h, :]
            s = lax.dot_general(
                qh, kh, (((1,), (1,)), ((), ())),
                preferred_element_type=jnp.float32)
            w = jnp.exp(s + bias)
            l = jnp.sum(w, axis=1, keepdims=True)
            o = jnp.dot(w, vh, preferred_element_type=jnp.float32)
            comm[0, :, h * DH:(h + 1) * DH] = o.astype(jnp.bfloat16)
            comm[0, :, DM + h:DM + h + 1] = l.astype(jnp.bfloat16)

        r0 = pltpu.make_async_remote_copy(
            src_ref=comm.at[0], dst_ref=comm.at[1],
            send_sem=ss.at[0], recv_sem=rs.at[0],
            device_id=(right,), device_id_type=pl.DeviceIdType.MESH,
        )
        l0 = pltpu.make_async_remote_copy(
            src_ref=comm.at[0], dst_ref=comm.at[2],
            send_sem=ss.at[1], recv_sem=rs.at[1],
            device_id=(left,), device_id_type=pl.DeviceIdType.MESH,
        )
        r0.start()
        l0.start()

        r0.wait_recv()
        r1 = pltpu.make_async_remote_copy(
            src_ref=comm.at[1, pl.ds(0, HALF)],
            dst_ref=comm.at[3, pl.ds(0, HALF)],
            send_sem=ss.at[2], recv_sem=rs.at[2],
            device_id=(right,), device_id_type=pl.DeviceIdType.MESH,
        )
        r1.start()

        l0.wait_recv()
        l1 = pltpu.make_async_remote_copy(
            src_ref=comm.at[2, pl.ds(HALF, HALF)],
            dst_ref=comm.at[3, pl.ds(HALF, HALF)],
            send_sem=ss.at[3], recv_sem=rs.at[3],
            device_id=(left,), device_id_type=pl.DeviceIdType.MESH,
        )
        l1.start()

        part = (comm[0, :, :].astype(jnp.float32)
                + comm[1, :, :].astype(jnp.float32)
                + comm[2, :, :].astype(jnp.float32))

        r1.wait_recv()
        l1.wait_recv()
        tot = part + comm[3, :, :].astype(jnp.float32)

        for h in range(HQ):
            ctx_ref[:, h * DH:(h + 1) * DH] = (
                tot[:, h * DH:(h + 1) * DH] / tot[:, DM + h:DM + h + 1])

        out_ref[0] = jnp.dot(ctx_ref[:, :], wo_ref[:, :],
                             preferred_element_type=jnp.float32)

        r0.wait_send()
        l0.wait_send()
        r1.wait_send()
        l1.wait_send()

    return pl.pallas_call(
        body,
        out_shape=jax.ShapeDtypeStruct((1, SQ, DM), jnp.float32),
        in_specs=[pl.BlockSpec(memory_space=pltpu.VMEM)] * 5,
        out_specs=pl.BlockSpec(memory_space=pltpu.VMEM),
        scratch_shapes=[
            pltpu.VMEM((N_DEV, SQ, CW), jnp.bfloat16),
            pltpu.VMEM((SQ, DM), jnp.float32),
            pltpu.SemaphoreType.DMA((4,)),
            pltpu.SemaphoreType.DMA((4,)),
        ],
        compiler_params=pltpu.CompilerParams(
            collective_id=0,
            vmem_limit_bytes=100 * 1024 * 1024,
        ),
    )(x, Wq, K_ext, V_ext, Wo)


# device time: 55107 ns/iter; 1.7991x vs baseline; 1.1546x over previous
import jax
import jax.numpy as jnp
from jax import lax
from jax.experimental import pallas as pl
from jax.experimental.pallas import tpu as pltpu

N_DEV = 4
SQ = 256
SKV_SHARD = 4096
HQ = 8
DH = 128
DM = HQ * DH
CW = DM + DH
SCALE = 0.08838834764831843
NEG = -1e9
HALF = SQ // 2


def kernel(x, Wq, K_ext, V_ext, Wo):
    def body(x_ref, wq_ref, k_ref, v_ref, wo_ref, out_ref,
             comm, ctx_ref, ss, rs):
        my_pos = lax.axis_index("i")
        left = lax.rem(my_pos + N_DEV - 1, N_DEV)
        right = lax.rem(my_pos + 1, N_DEV)

        barrier_sem = pltpu.get_barrier_semaphore()
        for nbr in (left, right):
            pl.semaphore_signal(
                barrier_sem, inc=1,
                device_id=(nbr,), device_id_type=pl.DeviceIdType.MESH,
            )
        pl.semaphore_wait(barrier_sem, 2)

        q = jnp.dot(x_ref[0], wq_ref[:, :],
                    preferred_element_type=jnp.float32) * SCALE

        for h in range(HQ):
            for j in range(4):
                qhj = q[j * 64:(j + 1) * 64, h * DH:(h + 1) * DH]
                khj = k_ref[0, :, j, :, h, :].reshape(SKV_SHARD // 4, DH)
                vhj = v_ref[0, :, j, :, h, :].reshape(SKV_SHARD // 4, DH)
                s = lax.dot_general(
                    qhj, khj, (((1,), (1,)), ((), ())),
                    preferred_element_type=jnp.float32)
                w = jnp.exp(s)
                l = jnp.sum(w, axis=1, keepdims=True)
                o = jnp.dot(w, vhj, preferred_element_type=jnp.float32)
                rows = pl.ds(j * 64, 64)
                comm[0, rows, h * DH:(h + 1) * DH] = o.astype(jnp.bfloat16)
                comm[0, rows, DM + h:DM + h + 1] = l.astype(jnp.bfloat16)

        r0 = pltpu.make_async_remote_copy(
            src_ref=comm.at[0], dst_ref=comm.at[1],
            send_sem=ss.at[0], recv_sem=rs.at[0],
            device_id=(right,), device_id_type=pl.DeviceIdType.MESH,
        )
        l0 = pltpu.make_async_remote_copy(
            src_ref=comm.at[0], dst_ref=comm.at[2],
            send_sem=ss.at[1], recv_sem=rs.at[1],
            device_id=(left,), device_id_type=pl.DeviceIdType.MESH,
        )
        r0.start()
        l0.start()

        r0.wait_recv()
        r1 = pltpu.make_async_remote_copy(
            src_ref=comm.at[1, pl.ds(0, HALF)],
            dst_ref=comm.at[3, pl.ds(0, HALF)],
            send_sem=ss.at[2], recv_sem=rs.at[2],
            device_id=(right,), device_id_type=pl.DeviceIdType.MESH,
        )
        r1.start()

        l0.wait_recv()
        l1 = pltpu.make_async_remote_copy(
            src_ref=comm.at[2, pl.ds(HALF, HALF)],
            dst_ref=comm.at[3, pl.ds(HALF, HALF)],
            send_sem=ss.at[3], recv_sem=rs.at[3],
            device_id=(left,), device_id_type=pl.DeviceIdType.MESH,
        )
        l1.start()

        part = (comm[0, :, :].astype(jnp.float32)
                + comm[1, :, :].astype(jnp.float32)
                + comm[2, :, :].astype(jnp.float32))

        r1.wait_recv()
        l1.wait_recv()
        tot = part + comm[3, :, :].astype(jnp.float32)

        for h in range(HQ):
            ctx_ref[:, h * DH:(h + 1) * DH] = (
                tot[:, h * DH:(h + 1) * DH] / tot[:, DM + h:DM + h + 1])

        out_ref[0] = jnp.dot(ctx_ref[:, :], wo_ref[:, :],
                             preferred_element_type=jnp.float32)

        r0.wait_send()
        l0.wait_send()
        r1.wait_send()
        l1.wait_send()

    return pl.pallas_call(
        body,
        out_shape=jax.ShapeDtypeStruct((1, SQ, DM), jnp.float32),
        in_specs=[pl.BlockSpec(memory_space=pltpu.VMEM)] * 5,
        out_specs=pl.BlockSpec(memory_space=pltpu.VMEM),
        scratch_shapes=[
            pltpu.VMEM((N_DEV, SQ, CW), jnp.bfloat16),
            pltpu.VMEM((SQ, DM), jnp.float32),
            pltpu.SemaphoreType.DMA((4,)),
            pltpu.SemaphoreType.DMA((4,)),
        ],
        compiler_params=pltpu.CompilerParams(
            collective_id=0,
            vmem_limit_bytes=100 * 1024 * 1024,
        ),
    )(x, Wq,
      K_ext.reshape(1, 16, 4, 64, HQ, DH),
      V_ext.reshape(1, 16, 4, 64, HQ, DH),
      Wo)
